# baseline (speedup 1.0000x reference)
import functools

import jax
import jax.numpy as jnp
from jax import lax
from jax.experimental import pallas as pl
from jax.experimental.pallas import tpu as pltpu
from jax.experimental.pallas import tpu_sc as plsc

_CH = 128
_Q = 4


def _sc_scores(node_i, node_j, neg_flat, table):
    info = plsc.get_sparse_core_info()
    nc, ns = info.num_cores, info.num_subcores
    nw = nc * ns
    b = node_i.shape[0]
    d = table.shape[1]
    nneg = neg_flat.shape[0]
    neg = nneg // b
    epw = b // nw
    eq = epw // _Q
    nsc = 1 + neg
    mesh = plsc.VectorSubcoreMesh(core_axis_name="c", subcore_axis_name="s")

    row_buf = lambda n: pltpu.VMEM((n, d), jnp.float32)

    @functools.partial(
        pl.kernel,
        mesh=mesh,
        out_type=jax.ShapeDtypeStruct((nsc, b), jnp.float32),
        scratch_types=[
            pltpu.VMEM((epw,), jnp.int32),
            pltpu.VMEM((epw,), jnp.int32),
            pltpu.VMEM((epw * neg,), jnp.int32),
            row_buf(eq), row_buf(eq), row_buf(eq * neg),
            row_buf(eq), row_buf(eq), row_buf(eq * neg),
            pltpu.VMEM((nsc, epw), jnp.float32),
            pltpu.SemaphoreType.DMA,
        ],
        compiler_params=pltpu.CompilerParams(
            use_tc_tiling_on_sc=False, needs_layout_passes=False
        ),
    )
    def fn(ni, nj, nn, tbl, out_s,
           idx_i, idx_j, idx_n, ri0, rj0, rn0, ri1, rj1, rn1, sv, sem):
        wid = lax.axis_index("s") * nc + lax.axis_index("c")
        sets = ((ri0, rj0, rn0), (ri1, rj1, rn1))
        iota16 = lax.iota(jnp.int32, 16)

        pltpu.sync_copy(ni.at[pl.ds(wid * epw, epw)], idx_i)
        pltpu.sync_copy(nj.at[pl.ds(wid * epw, epw)], idx_j)
        pltpu.sync_copy(nn.at[pl.ds(wid * epw * neg, epw * neg)], idx_n)

        def fire(qi, ri_b, rj_b, rn_b):
            ds_ = [
                pltpu.async_copy(tbl.at[idx_i.at[pl.ds(qi * eq, eq)]], ri_b, sem),
                pltpu.async_copy(tbl.at[idx_j.at[pl.ds(qi * eq, eq)]], rj_b, sem),
            ]
            for c in range(neg):
                ds_.append(pltpu.async_copy(
                    tbl.at[idx_n.at[pl.ds(qi * eq * neg + c * _CH, _CH)]],
                    rn_b.at[pl.ds(c * _CH, _CH)], sem))
            return ds_

        def compute(qi, ri_b, rj_b, rn_b):
            def group(g, carry):
                e = g * 16 + iota16
                e5 = e * neg
                acc = [jnp.zeros((16,), jnp.float32) for _ in range(nsc)]
                for dd in range(d):
                    cvec = jnp.full((16,), dd, jnp.int32)
                    ai = plsc.load_gather(ri_b, [e, cvec])
                    aj = plsc.load_gather(rj_b, [e, cvec])
                    acc[0] = acc[0] + ai * aj
                    for k in range(neg):
                        nk = plsc.load_gather(rn_b, [e5 + k, cvec])
                        acc[1 + k] = acc[1 + k] + ai * nk
                base = qi * eq + g * 16
                for r in range(nsc):
                    sv[r, pl.ds(base, 16)] = acc[r]
                return carry
            lax.fori_loop(0, eq // 16, group, 0)

        pend = fire(0, *sets[0])
        for qi in range(_Q):
            for dsc in pend:
                dsc.wait()
            pend = fire(qi + 1, *sets[(qi + 1) % 2]) if qi + 1 < _Q else []
            compute(qi, *sets[qi % 2])
        pltpu.sync_copy(sv, out_s.at[:, pl.ds(wid * epw, epw)])

    return fn(node_i, node_j, neg_flat, table)


def _tc_loss(scores):
    nsc, b = scores.shape

    def body(s_ref, o_ref):
        loss = -jnp.log(jax.nn.sigmoid(s_ref[0, :]))
        for k in range(1, nsc):
            loss = loss - jnp.log(1.0 - jax.nn.sigmoid(s_ref[k, :]))
        o_ref[...] = loss

    return pl.pallas_call(
        body,
        out_shape=jax.ShapeDtypeStruct((b,), jnp.float32),
    )(scores)


def kernel(node_i, node_j, neg_samples, table):
    ni = node_i.astype(jnp.int32)
    nj = node_j.astype(jnp.int32)
    nn = neg_samples.reshape(-1).astype(jnp.int32)
    scores = _sc_scores(ni, nj, nn, table)
    return _tc_loss(scores)


# speedup vs baseline: 1.0552x; 1.0552x over previous
import functools

import jax
import jax.numpy as jnp
from jax import lax
from jax.experimental import pallas as pl
from jax.experimental.pallas import tpu as pltpu
from jax.experimental.pallas import tpu_sc as plsc

_CH = 128
_NO = 8


def _sc_partials(node_i, node_j, neg_flat, table):
    info = plsc.get_sparse_core_info()
    nc, ns = info.num_cores, info.num_subcores
    nw = nc * ns
    b = node_i.shape[0]
    d = table.shape[1]
    nneg = neg_flat.shape[0]
    neg = nneg // b
    epw = b // nw
    eo = epw // _NO
    nsc = 1 + neg
    mesh = plsc.VectorSubcoreMesh(core_axis_name="c", subcore_axis_name="s")

    row_buf = lambda n: pltpu.VMEM((n, d), jnp.float32)

    @functools.partial(
        pl.kernel,
        mesh=mesh,
        out_type=jax.ShapeDtypeStruct((nsc, b * 16), jnp.float32),
        scratch_types=[
            pltpu.VMEM((epw,), jnp.int32),
            pltpu.VMEM((epw,), jnp.int32),
            pltpu.VMEM((epw * neg,), jnp.int32),
            row_buf(eo), row_buf(eo), row_buf(eo * neg),
            row_buf(eo), row_buf(eo), row_buf(eo * neg),
            pltpu.VMEM((nsc, eo * 16), jnp.float32),
            pltpu.VMEM((nsc, eo * 16), jnp.float32),
            pltpu.SemaphoreType.DMA,
            pltpu.SemaphoreType.DMA,
        ],
        compiler_params=pltpu.CompilerParams(use_tc_tiling_on_sc=False),
    )
    def fn(ni, nj, nn, tbl, out_p,
           idx_i, idx_j, idx_n, ri0, rj0, rn0, ri1, rj1, rn1, pb0, pb1,
           sem, osem):
        wid = lax.axis_index("s") * nc + lax.axis_index("c")
        sets = ((ri0, rj0, rn0, pb0), (ri1, rj1, rn1, pb1))

        pltpu.sync_copy(ni.at[pl.ds(wid * epw, epw)], idx_i)
        pltpu.sync_copy(nj.at[pl.ds(wid * epw, epw)], idx_j)
        pltpu.sync_copy(nn.at[pl.ds(wid * epw * neg, epw * neg)], idx_n)

        def fire(oi, ri_b, rj_b, rn_b):
            ds_ = [
                pltpu.async_copy(tbl.at[idx_i.at[pl.ds(oi * eo, eo)]], ri_b, sem),
                pltpu.async_copy(tbl.at[idx_j.at[pl.ds(oi * eo, eo)]], rj_b, sem),
            ]
            nbase = oi * eo * neg
            nrows = eo * neg
            for c0 in range(0, nrows, _CH):
                cn = min(_CH, nrows - c0)
                ds_.append(pltpu.async_copy(
                    tbl.at[idx_n.at[pl.ds(nbase + c0, cn)]],
                    rn_b.at[pl.ds(c0, cn)], sem))
            return ds_

        def compute(ri_b, rj_b, rn_b, pb):
            def elem(e, carry):
                ai = [ri_b[e, pl.ds(16 * p, 16)] for p in range(d // 16)]

                def dot_to(row_ref, r, out_row):
                    acc = ai[0] * row_ref[r, pl.ds(0, 16)]
                    for p in range(1, d // 16):
                        acc = acc + ai[p] * row_ref[r, pl.ds(16 * p, 16)]
                    pb[out_row, pl.ds(e * 16, 16)] = acc

                dot_to(rj_b, e, 0)
                for k in range(neg):
                    dot_to(rn_b, e * neg + k, 1 + k)
                return carry
            lax.fori_loop(0, eo, elem, 0)

        pend = fire(0, *sets[0][:3])
        out_pend = []
        for oi in range(_NO):
            cur = sets[oi % 2]
            for dsc in pend:
                dsc.wait()
            pend = fire(oi + 1, *sets[(oi + 1) % 2][:3]) if oi + 1 < _NO else []
            for dsc in out_pend:
                dsc.wait()
            compute(*cur)
            out_pend = [pltpu.async_copy(
                cur[3],
                out_p.at[:, pl.ds((wid * epw + oi * eo) * 16, eo * 16)],
                osem)]
        for dsc in out_pend:
            dsc.wait()

    return fn(node_i, node_j, neg_flat, table)


def _tc_loss(partials, b):
    nsc = partials.shape[0]
    bb = 2048

    def body(p_ref, o_ref):
        p = p_ref[...].reshape(nsc, bb, 16)
        s = jnp.sum(p, axis=-1)
        loss = -jnp.log(jax.nn.sigmoid(s[0]))
        for k in range(1, nsc):
            loss = loss - jnp.log(1.0 - jax.nn.sigmoid(s[k]))
        o_ref[...] = loss

    return pl.pallas_call(
        body,
        grid=(b // bb,),
        in_specs=[pl.BlockSpec((nsc, bb * 16), lambda i: (0, i))],
        out_specs=pl.BlockSpec((bb,), lambda i: (i,)),
        out_shape=jax.ShapeDtypeStruct((b,), jnp.float32),
    )(partials)


def kernel(node_i, node_j, neg_samples, table):
    b = node_i.shape[0]
    ni = node_i.astype(jnp.int32)
    nj = node_j.astype(jnp.int32)
    nn = neg_samples.reshape(-1).astype(jnp.int32)
    partials = _sc_partials(ni, nj, nn, table)
    return _tc_loss(partials, b)


# speedup vs baseline: 1.4793x; 1.4019x over previous
import functools

import jax
import jax.numpy as jnp
from jax import lax
from jax.experimental import pallas as pl
from jax.experimental.pallas import tpu as pltpu
from jax.experimental.pallas import tpu_sc as plsc

_CH = 128
_TCB = 4096


def _tc_transpose(tbl_t):
    d, v = tbl_t.shape
    nblk = -(-v // _TCB)

    def body(x_ref, o_ref):
        y = jnp.transpose(x_ref[...], (1, 0))
        o_ref[...] = jnp.concatenate([y, y], axis=1)

    return pl.pallas_call(
        body,
        grid=(nblk,),
        in_specs=[pl.BlockSpec((d, _TCB), lambda i: (0, i))],
        out_specs=pl.BlockSpec((_TCB, 2 * d), lambda i: (i, 0)),
        out_shape=jax.ShapeDtypeStruct((nblk * _TCB, 2 * d), jnp.float32),
    )(tbl_t)


def _sc_partials(node_i, node_j, neg_flat, table2):
    info = plsc.get_sparse_core_info()
    nc, ns = info.num_cores, info.num_subcores
    nw = nc * ns
    b = node_i.shape[0]
    d2 = table2.shape[1]
    d = d2 // 2
    nneg = neg_flat.shape[0]
    neg = nneg // b
    epw = b // nw
    nch = 16
    eo = epw // nch
    nsc = 1 + neg
    mesh = plsc.VectorSubcoreMesh(core_axis_name="c", subcore_axis_name="s")

    row_buf = lambda n: pltpu.VMEM((n, d2), jnp.float32)

    @functools.partial(
        pl.kernel,
        mesh=mesh,
        out_type=jax.ShapeDtypeStruct((nsc, b * 16), jnp.float32),
        scratch_types=[
            pltpu.VMEM((epw,), jnp.int32),
            pltpu.VMEM((epw,), jnp.int32),
            pltpu.VMEM((epw * neg,), jnp.int32),
            row_buf(eo), row_buf(eo), row_buf(eo * neg),
            row_buf(eo), row_buf(eo), row_buf(eo * neg),
            pltpu.VMEM((nsc, eo * 16), jnp.float32),
            pltpu.VMEM((nsc, eo * 16), jnp.float32),
            pltpu.SemaphoreType.DMA,
            pltpu.SemaphoreType.DMA,
        ],
    )
    def fn(ni, nj, nn, tbl, out_p,
           idx_i, idx_j, idx_n, ri0, rj0, rn0, ri1, rj1, rn1, pb0, pb1,
           sem, osem):
        wid = lax.axis_index("s") * nc + lax.axis_index("c")
        sets = ((ri0, rj0, rn0, pb0), (ri1, rj1, rn1, pb1))

        pltpu.sync_copy(ni.at[pl.ds(wid * epw, epw)], idx_i)
        pltpu.sync_copy(nj.at[pl.ds(wid * epw, epw)], idx_j)
        pltpu.sync_copy(nn.at[pl.ds(wid * epw * neg, epw * neg)], idx_n)

        def fire(oi, ri_b, rj_b, rn_b):
            ds_ = [
                pltpu.async_copy(tbl.at[idx_i.at[pl.ds(oi * eo, eo)]], ri_b, sem),
                pltpu.async_copy(tbl.at[idx_j.at[pl.ds(oi * eo, eo)]], rj_b, sem),
            ]
            nbase = oi * eo * neg
            nrows = eo * neg
            for c0 in range(0, nrows, _CH):
                cn = min(_CH, nrows - c0)
                ds_.append(pltpu.async_copy(
                    tbl.at[idx_n.at[pl.ds(nbase + c0, cn)]],
                    rn_b.at[pl.ds(c0, cn)], sem))
            return ds_

        def compute(ri_b, rj_b, rn_b, pb):
            def elem(e, carry):
                a = [ri_b[e, pl.ds(16 * p, 16)] for p in range(d // 16)]

                def dot_to(row_ref, r, out_row):
                    acc = a[0] * row_ref[r, pl.ds(0, 16)]
                    for p in range(1, d // 16):
                        acc = acc + a[p] * row_ref[r, pl.ds(16 * p, 16)]
                    pb[out_row, pl.ds(e * 16, 16)] = acc

                dot_to(rj_b, e, 0)
                for k in range(neg):
                    dot_to(rn_b, e * neg + k, 1 + k)
                return carry
            lax.fori_loop(0, eo, elem, 0)

        pend = fire(0, *sets[0][:3])
        out_pend = []
        for oi in range(nch):
            cur = sets[oi % 2]
            for dsc in pend:
                dsc.wait()
            pend = fire(oi + 1, *sets[(oi + 1) % 2][:3]) if oi + 1 < nch else []
            for dsc in out_pend:
                dsc.wait()
            compute(*cur)
            out_pend = [pltpu.async_copy(
                cur[3],
                out_p.at[:, pl.ds((wid * epw + oi * eo) * 16, eo * 16)],
                osem)]
        for dsc in out_pend:
            dsc.wait()

    return fn(node_i, node_j, neg_flat, table2)


def _tc_loss(partials, b):
    nsc = partials.shape[0]
    bb = 2048

    def body(p_ref, o_ref):
        p = p_ref[...].reshape(nsc, bb, 16)
        s = jnp.sum(p, axis=-1)
        loss = -jnp.log(jax.nn.sigmoid(s[0]))
        for k in range(1, nsc):
            loss = loss - jnp.log(1.0 - jax.nn.sigmoid(s[k]))
        o_ref[...] = loss

    return pl.pallas_call(
        body,
        grid=(b // bb,),
        in_specs=[pl.BlockSpec((nsc, bb * 16), lambda i: (0, i))],
        out_specs=pl.BlockSpec((bb,), lambda i: (i,)),
        out_shape=jax.ShapeDtypeStruct((b,), jnp.float32),
    )(partials)


def kernel(node_i, node_j, neg_samples, table):
    b = node_i.shape[0]
    ni = node_i.astype(jnp.int32)
    nj = node_j.astype(jnp.int32)
    nn = neg_samples.reshape(-1).astype(jnp.int32)
    table2 = _tc_transpose(table.T)
    partials = _sc_partials(ni, nj, nn, table2)
    return _tc_loss(partials, b)
